# Initial kernel scaffold; baseline (speedup 1.0000x reference)
import jax
import jax.numpy as jnp
from jax.experimental import pallas as pl


def kernel(x, edge_index, W1, b1, W_ih, W_hh, b_ih, b_hh, W2, b2):
    raise NotImplementedError("write your pallas kernel here")


# speedup vs baseline: 6.6700x; 6.6700x over previous
import functools

import jax
import jax.numpy as jnp
from jax import lax
from jax.experimental import pallas as pl
from jax.experimental.pallas import tpu as pltpu
from jax.experimental.pallas import tpu_sc as plsc

N = 10000
E = 160000
D = 256
H3 = 768

NC = 2
NS = 16
L = 16

_MESH = dict(core_axis_name="c", subcore_axis_name="s",
             num_cores=NC, num_subcores=NS)


def _zero_chunks(ref, n):
  z = jnp.zeros((L,), jnp.float32)
  def body(j, _):
    ref[pl.ds(j * L, L)] = z
    return 0
  lax.fori_loop(0, n // L, body, 0)


NPAD = 10240
EPAD = 163840
EPWA = EPAD // (NC * NS)
CHA = 64
SPT = NPAD // NS


def _deg_body(dst_hbm, deg_hbm, dst_i, ones_v, zero_v, acc_sh):
  c = lax.axis_index("c")
  s = lax.axis_index("s")

  one = jnp.ones((L,), jnp.float32)
  def fill(j, _):
    ones_v[pl.ds(j * L, L)] = one
    return 0
  lax.fori_loop(0, CHA // L, fill, 0)
  _zero_chunks(zero_v, SPT)
  pltpu.sync_copy(zero_v, acc_sh.at[pl.ds(s * SPT, SPT)])
  plsc.subcore_barrier()

  def chunk(ci, _):
    off = c * (EPAD // NC) + s * EPWA + ci * CHA
    pltpu.sync_copy(dst_hbm.at[pl.ds(off, CHA)], dst_i)
    pltpu.sync_copy(ones_v, acc_sh.at[dst_i], add=True)
    return 0
  lax.fori_loop(0, EPWA // CHA, chunk, 0)

  plsc.subcore_barrier()
  pltpu.sync_copy(acc_sh.at[pl.ds(s * SPT, SPT)],
                  deg_hbm.at[c, pl.ds(s * SPT, SPT)])


def _deg(dst):
  return pl.kernel(
      _deg_body,
      out_type=jax.ShapeDtypeStruct((NC, NPAD), jnp.float32),
      mesh=plsc.VectorSubcoreMesh(**_MESH),
      scratch_types=[
          pltpu.VMEM((CHA,), jnp.int32),
          pltpu.VMEM((CHA,), jnp.float32),
          pltpu.VMEM((SPT,), jnp.float32),
          pltpu.VMEM_SHARED((NPAD,), jnp.float32),
      ],
  )(dst)


HD = D // 2
CHE = 16
ACC1 = NPAD * HD
ZCH = 5120


def _agg_body(src_hbm, dst_hbm, hp_hbm, out_hbm,
              src_i, dst_i, sidx, rows_v, zero_v, acc_sh, sem, sem2):
  c = lax.axis_index("c")
  s = lax.axis_index("s")
  iotas = [lax.iota(jnp.int32, L) + L * m for m in range(HD // L)]

  _zero_chunks(zero_v, ZCH)
  for p in range(2):
    if p:
      plsc.subcore_barrier()
    for r in range(ACC1 // NS // ZCH):
      pltpu.sync_copy(zero_v,
                      acc_sh.at[pl.ds((s * (ACC1 // NS) + r * ZCH), ZCH)])
    plsc.subcore_barrier()

    def chunk(ci, _):
      off = c * (EPAD // NC) + s * EPWA + ci * CHE
      pltpu.sync_copy(src_hbm.at[pl.ds(off, CHE)], src_i)
      pltpu.sync_copy(dst_hbm.at[pl.ds(off, CHE)], dst_i)
      gh = pltpu.async_copy(hp_hbm.at[src_i], rows_v, sem)
      dv = dst_i[...]
      for j in range(CHE):
        sb = dv[j] * HD
        for m in range(HD // L):
          sidx[j, pl.ds(m * L, L)] = iotas[m] + sb
      gh.wait()
      sh = [pltpu.async_copy(rows_v.at[j, pl.ds(p * HD, HD)],
                             acc_sh.at[sidx.at[j]], sem2, add=True)
            for j in range(CHE)]
      for h in sh:
        h.wait()
      return 0
    lax.fori_loop(0, EPWA // CHE, chunk, 0)

    plsc.subcore_barrier()
    pltpu.sync_copy(acc_sh.at[pl.ds(s * (ACC1 // NS), ACC1 // NS)],
                    out_hbm.at[c, p, pl.ds(s * (ACC1 // NS), ACC1 // NS)])


def _agg(src, dst, hp):
  return pl.kernel(
      _agg_body,
      out_type=jax.ShapeDtypeStruct((NC, 2, ACC1), jnp.float32),
      mesh=plsc.VectorSubcoreMesh(**_MESH),
      scratch_types=[
          pltpu.VMEM((CHE,), jnp.int32),
          pltpu.VMEM((CHE,), jnp.int32),
          pltpu.VMEM((CHE, HD), jnp.int32),
          pltpu.VMEM((CHE, D), jnp.float32),
          pltpu.VMEM((ZCH,), jnp.float32),
          pltpu.VMEM_SHARED((ACC1,), jnp.float32),
          pltpu.SemaphoreType.DMA,
          pltpu.SemaphoreType.DMA,
      ],
  )(src, dst, hp)


def _fin_body(src_hbm, dst_hbm, h2p_hbm, out_hbm,
              src_i, dst_i, vals_v, zero_v, acc_sh, sem):
  c = lax.axis_index("c")
  s = lax.axis_index("s")

  _zero_chunks(zero_v, SPT)
  pltpu.sync_copy(zero_v, acc_sh.at[pl.ds(s * SPT, SPT)])
  plsc.subcore_barrier()

  def chunk(ci, _):
    off = c * (EPAD // NC) + s * EPWA + ci * CHA
    pltpu.sync_copy(src_hbm.at[pl.ds(off, CHA)], src_i)
    pltpu.sync_copy(dst_hbm.at[pl.ds(off, CHA)], dst_i)
    pltpu.async_copy(h2p_hbm.at[src_i], vals_v, sem).wait()
    pltpu.sync_copy(vals_v, acc_sh.at[dst_i], add=True)
    return 0
  lax.fori_loop(0, EPWA // CHA, chunk, 0)

  plsc.subcore_barrier()
  pltpu.sync_copy(acc_sh.at[pl.ds(s * SPT, SPT)],
                  out_hbm.at[c, pl.ds(s * SPT, SPT)])


def _fin(src, dst, h2p):
  return pl.kernel(
      _fin_body,
      out_type=jax.ShapeDtypeStruct((NC, NPAD), jnp.float32),
      mesh=plsc.VectorSubcoreMesh(**_MESH),
      scratch_types=[
          pltpu.VMEM((CHA,), jnp.int32),
          pltpu.VMEM((CHA,), jnp.int32),
          pltpu.VMEM((CHA,), jnp.float32),
          pltpu.VMEM((SPT,), jnp.float32),
          pltpu.VMEM_SHARED((NPAD,), jnp.float32),
          pltpu.SemaphoreType.DMA,
      ],
  )(src, dst, h2p)


def _fincomb_body(fa_ref, fb_ref, h2p_ref, dinv_ref, b2_ref, out_ref):
  out_ref[...] = (dinv_ref[...] * (fa_ref[...] + fb_ref[...] + h2p_ref[...])
                  + b2_ref[...])


def _fincomb(fa, fb, h2p, dinv, b2):
  return pl.pallas_call(
      _fincomb_body,
      grid=(GRID,),
      in_specs=[
          pl.BlockSpec((RB, 1), lambda i: (i, 0)),
          pl.BlockSpec((RB, 1), lambda i: (i, 0)),
          pl.BlockSpec((RB, 1), lambda i: (i, 0)),
          pl.BlockSpec((RB, 1), lambda i: (i, 0)),
          pl.BlockSpec((1, 1), lambda i: (0, 0)),
      ],
      out_specs=pl.BlockSpec((RB, 1), lambda i: (i, 0)),
      out_shape=jax.ShapeDtypeStruct((N, 1), jnp.float32),
  )(fa, fb, h2p, dinv, b2)


RB = 400
GRID = N // RB


def _lin1_body(x_ref, w1_ref, dega_ref, degb_ref, hp_ref, dinv_ref):
  deg = dega_ref[...] + degb_ref[...] + 1.0
  di = lax.rsqrt(jnp.maximum(deg, 1.0))
  h = jnp.dot(x_ref[...], w1_ref[...], preferred_element_type=jnp.float32)
  hp_ref[...] = di * h
  dinv_ref[...] = di


def _lin1(x, W1, dega, degb):
  return pl.pallas_call(
      _lin1_body,
      grid=(GRID,),
      in_specs=[
          pl.BlockSpec((RB, D), lambda i: (i, 0)),
          pl.BlockSpec((D, D), lambda i: (0, 0)),
          pl.BlockSpec((RB, 1), lambda i: (i, 0)),
          pl.BlockSpec((RB, 1), lambda i: (i, 0)),
      ],
      out_specs=[
          pl.BlockSpec((RB, D), lambda i: (i, 0)),
          pl.BlockSpec((RB, 1), lambda i: (i, 0)),
      ],
      out_shape=[
          jax.ShapeDtypeStruct((N, D), jnp.float32),
          jax.ShapeDtypeStruct((N, 1), jnp.float32),
      ],
  )(x, W1, dega, degb)


def _gates_body(agga_ref, aggb_ref, hp_ref, dinv_ref, b1_ref, wih_ref,
                bih_ref, gi_ref):
  out1 = jax.nn.relu(
      dinv_ref[...] * (agga_ref[...] + aggb_ref[...] + hp_ref[...])
      + b1_ref[...])
  gi_ref[...] = lax.dot_general(
      out1, wih_ref[...], (((1,), (1,)), ((), ())),
      preferred_element_type=jnp.float32) + bih_ref[...]


def _gates(agga, aggb, hp, dinv, b1, W_ih, b_ih):
  return pl.pallas_call(
      _gates_body,
      grid=(GRID,),
      in_specs=[
          pl.BlockSpec((RB, D), lambda i: (i, 0)),
          pl.BlockSpec((RB, D), lambda i: (i, 0)),
          pl.BlockSpec((RB, D), lambda i: (i, 0)),
          pl.BlockSpec((RB, 1), lambda i: (i, 0)),
          pl.BlockSpec((1, D), lambda i: (0, 0)),
          pl.BlockSpec((H3, D), lambda i: (0, 0)),
          pl.BlockSpec((1, H3), lambda i: (0, 0)),
      ],
      out_specs=pl.BlockSpec((RB, H3), lambda i: (i, 0)),
      out_shape=jax.ShapeDtypeStruct((N, H3), jnp.float32),
  )(agga, aggb, hp, dinv, b1, W_ih, b_ih)


def _gru_body(gi_ref, whh_ref, bhh_ref, w2_ref, dinv_ref, out_ref,
              h_s, y_s):
  @pl.when(pl.program_id(0) == 0)
  def _():
    h_s[...] = jnp.zeros((1, D), jnp.float32)

  def step(t, _):
    h = h_s[...]
    gi = gi_ref[pl.ds(t, 1), :]
    gh = lax.dot_general(h, whh_ref[...], (((1,), (1,)), ((), ())),
                         preferred_element_type=jnp.float32) + bhh_ref[...]
    r = jax.nn.sigmoid(gi[:, 0:D] + gh[:, 0:D])
    z = jax.nn.sigmoid(gi[:, D:2 * D] + gh[:, D:2 * D])
    ng = jnp.tanh(gi[:, 2 * D:] + r * gh[:, 2 * D:])
    hn = (1.0 - z) * ng + z * h
    h_s[...] = hn
    y_s[pl.ds(t, 1), :] = hn
    return 0
  lax.fori_loop(0, RB, step, 0)

  out_ref[...] = dinv_ref[...] * jnp.dot(
      y_s[...], w2_ref[...], preferred_element_type=jnp.float32)


def _gru(GI, W_hh, b_hh, W2, dinv):
  return pl.pallas_call(
      _gru_body,
      grid=(GRID,),
      in_specs=[
          pl.BlockSpec((RB, H3), lambda i: (i, 0)),
          pl.BlockSpec((H3, D), lambda i: (0, 0)),
          pl.BlockSpec((1, H3), lambda i: (0, 0)),
          pl.BlockSpec((D, 1), lambda i: (0, 0)),
          pl.BlockSpec((RB, 1), lambda i: (i, 0)),
      ],
      out_specs=pl.BlockSpec((RB, 1), lambda i: (i, 0)),
      out_shape=jax.ShapeDtypeStruct((N, 1), jnp.float32),
      scratch_shapes=[
          pltpu.VMEM((1, D), jnp.float32),
          pltpu.VMEM((RB, D), jnp.float32),
      ],
  )(GI, W_hh, b_hh, W2, dinv)


@jax.jit
def kernel(x, edge_index, W1, b1, W_ih, W_hh, b_ih, b_hh, W2, b2):
  src = edge_index[0]
  dst = edge_index[1]
  pad = EPAD - E
  src_p = jnp.concatenate([src, jnp.zeros((pad,), jnp.int32)])
  dst_p = jnp.concatenate([dst, jnp.full((pad,), N, jnp.int32)])
  deg2 = _deg(dst_p)
  hp, dinv = _lin1(x, W1, deg2[0, :N, None], deg2[1, :N, None])
  agg2 = _agg(src_p, dst_p, hp)
  agg_a = jnp.concatenate([agg2[0, 0].reshape(NPAD, HD)[:N],
                           agg2[0, 1].reshape(NPAD, HD)[:N]], axis=1)
  agg_b = jnp.concatenate([agg2[1, 0].reshape(NPAD, HD)[:N],
                           agg2[1, 1].reshape(NPAD, HD)[:N]], axis=1)
  GI = _gates(agg_a, agg_b, hp, dinv,
              b1.reshape(1, D), W_ih, b_ih.reshape(1, H3))
  h2p = _gru(GI, W_hh, b_hh.reshape(1, H3), W2, dinv)
  fin2 = _fin(src_p, dst_p, h2p.reshape(N))
  out = _fincomb(fin2[0, :N, None], fin2[1, :N, None], h2p, dinv,
                 b2.reshape(1, 1))
  return out.reshape(N)
